# baseline (speedup 1.0000x reference)
import dataclasses
import functools

import jax
import jax.numpy as jnp
from jax import lax
from jax.experimental import pallas as pl
from jax.experimental.pallas import tpu as pltpu
from jax.experimental.pallas import tpu_sc as plsc

N = 10000
E = 320000
IN_CH = 128
HID = 8
HEADS = 8
OUT_CH = 16

NC = 2
NS = 16
LANES = 16

EB = 128
NW = NC * NS
E_TOT = E + N
BLOCKS_PER_W = 82
E_PAD = BLOCKS_PER_W * NW * EB
ROWS_PER_SUB = 632
NP = ROWS_PER_SUB * NS
DUMMY = N

_vmesh = plsc.VectorSubcoreMesh(
    core_axis_name="c", subcore_axis_name="s", num_cores=NC, num_subcores=NS)

_sc_params = pltpu.CompilerParams(use_tc_tiling_on_sc=False)
if "needs_layout_passes" in pltpu.CompilerParams.__dataclass_fields__:
    _sc_params = dataclasses.replace(_sc_params, needs_layout_passes=False)


def _make_sc_edge(wp, wa, npairs, expand, name):
    @functools.partial(
        pl.kernel,
        out_type=jax.ShapeDtypeStruct((NC, NP, wp), jnp.float32),
        mesh=_vmesh,
        scratch_types=[
            pltpu.VMEM((BLOCKS_PER_W, EB), jnp.int32),
            pltpu.VMEM((BLOCKS_PER_W, EB), jnp.int32),
            pltpu.VMEM((EB, wp), jnp.float32),
            pltpu.VMEM((EB, wa), jnp.float32),
            pltpu.VMEM((EB, wp), jnp.float32),
            pltpu.VMEM((EB, wa), jnp.float32),
            pltpu.VMEM((EB, wp), jnp.float32),
            pltpu.VMEM((EB, wp), jnp.float32),
            pltpu.VMEM_SHARED((NP, wp), jnp.float32),
            pltpu.SemaphoreType.DMA,
            pltpu.SemaphoreType.DMA,
        ],
        compiler_params=_sc_params,
        name=name,
    )
    def sc_edge(src_h, dst_h, p_h, a_h, out_h,
                src_i, dst_i, prow0, arow0, prow1, arow1, pay0, pay1,
                accum, sem0, sem1):
        cid = lax.axis_index("c")
        sid = lax.axis_index("s")
        wid = cid * NS + sid

        @pl.loop(0, EB)
        def _(r):
            for k in range(wp // LANES):
                prow0[r, pl.ds(k * LANES, LANES)] = jnp.zeros((LANES,),
                                                              jnp.float32)

        base_row = sid * ROWS_PER_SUB
        for off, nrows in ((0, 128), (128, 128), (256, 128), (384, 128),
                           (512, 120)):
            pltpu.sync_copy(prow0.at[pl.ds(0, nrows)],
                            accum.at[pl.ds(base_row + off, nrows)])
        plsc.subcore_barrier()

        eid = (NW - 1) - wid
        pltpu.sync_copy(src_h.at[pl.ds(eid * BLOCKS_PER_W, BLOCKS_PER_W)],
                        src_i)
        pltpu.sync_copy(dst_h.at[pl.ds(eid * BLOCKS_PER_W, BLOCKS_PER_W)],
                        dst_i)

        score_off = npairs * LANES
        lane = lax.iota(jnp.int32, LANES) // 8
        lanes = [lane + 2 * p for p in range(npairs)]

        def compute(prow, arow, pay):
            @plsc.parallel_loop(0, EB, unroll=4)
            def _(e):
                s = (prow[e, pl.ds(score_off, LANES)]
                     + arow[e, pl.ds(0, LANES)])
                s = jnp.maximum(s, 0.2 * s)
                w = jnp.exp(s)
                pay[e, pl.ds(score_off, LANES)] = w
                if expand:
                    for p in range(npairs):
                        wexp = plsc.load_gather(
                            pay.at[e], [lanes[p] + score_off])
                        pay[e, pl.ds(p * LANES, LANES)] = (
                            prow[e, pl.ds(p * LANES, LANES)] * wexp)
                else:
                    for p in range(npairs):
                        pay[e, pl.ds(p * LANES, LANES)] = (
                            prow[e, pl.ds(p * LANES, LANES)] * w)

        def gather_start(b, prow, arow, sem):
            pltpu.async_copy(p_h.at[src_i.at[b]], prow, sem)
            pltpu.async_copy(a_h.at[dst_i.at[b]], arow, sem)

        def gather_wait(b, prow, arow, sem):
            pltpu.make_async_copy(p_h.at[src_i.at[b]], prow, sem).wait()
            pltpu.make_async_copy(a_h.at[dst_i.at[b]], arow, sem).wait()

        def halfstep(b, prow, arow, pay, sem):
            gather_wait(b, prow, arow, sem)
            compute(prow, arow, pay)
            pltpu.sync_copy(pay, accum.at[dst_i.at[b]], add=True)
            gather_start(jnp.minimum(b + 2, BLOCKS_PER_W - 1), prow, arow,
                         sem)

        gather_start(0, prow0, arow0, sem0)
        gather_start(1, prow1, arow1, sem1)

        @pl.loop(0, BLOCKS_PER_W, step=2)
        def _(b):
            halfstep(b, prow0, arow0, pay0, sem0)
            halfstep(b + 1, prow1, arow1, pay1, sem1)

        gather_wait(0, prow0, arow0, sem0)
        gather_wait(1, prow1, arow1, sem1)

        plsc.subcore_barrier()
        pltpu.sync_copy(accum.at[pl.ds(base_row, ROWS_PER_SUB)],
                        out_h.at[cid, pl.ds(base_row, ROWS_PER_SUB)])

    return sc_edge


_sc_edge_l1 = _make_sc_edge(wp=80, wa=16, npairs=4, expand=True,
                            name="sc_gat_edges_l1")
_sc_edge_l2 = _make_sc_edge(wp=32, wa=16, npairs=1, expand=False,
                            name="sc_gat_edges_l2")


_RB = 1000
_GRID = N // _RB


def _tc1_body(x_ref, w_ref, asr_ref, adr_ref, p_ref, a_ref):
    h = jnp.dot(x_ref[...], w_ref[...], preferred_element_type=jnp.float32)
    hr = h.reshape(_RB, HEADS, HID)
    asrc = (hr * asr_ref[...][None]).sum(-1)
    adst = (hr * adr_ref[...][None]).sum(-1)
    zero8 = jnp.zeros((_RB, 8), jnp.float32)
    p_ref[...] = jnp.concatenate([h, asrc, zero8], axis=1)
    a_ref[...] = jnp.concatenate([adst, zero8], axis=1)


def _tc1(x, W1, as1, ad1):
    return pl.pallas_call(
        _tc1_body,
        grid=(_GRID,),
        in_specs=[
            pl.BlockSpec((_RB, IN_CH), lambda i: (i, 0)),
            pl.BlockSpec((IN_CH, HEADS * HID), lambda i: (0, 0)),
            pl.BlockSpec((HEADS, HID), lambda i: (0, 0)),
            pl.BlockSpec((HEADS, HID), lambda i: (0, 0)),
        ],
        out_specs=[
            pl.BlockSpec((_RB, 80), lambda i: (i, 0)),
            pl.BlockSpec((_RB, 16), lambda i: (i, 0)),
        ],
        out_shape=[
            jax.ShapeDtypeStruct((N, 80), jnp.float32),
            jax.ShapeDtypeStruct((N, 16), jnp.float32),
        ],
        name="tc_gat_dense1",
    )(x, W1, as1, ad1)


def _tc2_body(acc_ref, b1_ref, w2_ref, as2_ref, ad2_ref, p_ref, a_ref):
    acc = acc_ref[0] + acc_ref[1]
    den = acc[:, 64:72]
    msg = acc[:, 0:64].reshape(_RB, HEADS, HID)
    o1 = msg / (den[:, :, None] + 1e-16)
    z = o1.reshape(_RB, HEADS * HID) + b1_ref[...][None]
    act = jnp.where(z > 0, z, jnp.exp(jnp.minimum(z, 0.0)) - 1.0)
    h2 = jnp.dot(act, w2_ref[...], preferred_element_type=jnp.float32)
    asrc2 = (h2 * as2_ref[...]).sum(-1, keepdims=True)
    adst2 = (h2 * ad2_ref[...]).sum(-1, keepdims=True)
    p_ref[...] = jnp.concatenate(
        [h2, jnp.broadcast_to(asrc2, (_RB, 16))], axis=1)
    a_ref[...] = jnp.broadcast_to(adst2, (_RB, 16))


def _tc2(acc1, b1, W2, as2, ad2):
    return pl.pallas_call(
        _tc2_body,
        grid=(_GRID,),
        in_specs=[
            pl.BlockSpec((2, _RB, 80), lambda i: (0, i, 0)),
            pl.BlockSpec((HEADS * HID,), lambda i: (0,)),
            pl.BlockSpec((HEADS * HID, OUT_CH), lambda i: (0, 0)),
            pl.BlockSpec((1, OUT_CH), lambda i: (0, 0)),
            pl.BlockSpec((1, OUT_CH), lambda i: (0, 0)),
        ],
        out_specs=[
            pl.BlockSpec((_RB, 32), lambda i: (i, 0)),
            pl.BlockSpec((_RB, 16), lambda i: (i, 0)),
        ],
        out_shape=[
            jax.ShapeDtypeStruct((N, 32), jnp.float32),
            jax.ShapeDtypeStruct((N, 16), jnp.float32),
        ],
        name="tc_gat_dense2",
    )(acc1, b1, W2, as2, ad2)


def _tc3_body(acc_ref, b2_ref, o_ref):
    acc = acc_ref[0] + acc_ref[1]
    den = acc[:, 16:17]
    o = acc[:, 0:16] / (den + 1e-16) + b2_ref[...][None]
    m = o.max(axis=1, keepdims=True)
    s = o - m
    o_ref[...] = s - jnp.log(jnp.sum(jnp.exp(s), axis=1, keepdims=True))


def _tc3(acc2, b2):
    return pl.pallas_call(
        _tc3_body,
        grid=(_GRID,),
        in_specs=[
            pl.BlockSpec((2, _RB, 32), lambda i: (0, i, 0)),
            pl.BlockSpec((OUT_CH,), lambda i: (0,)),
        ],
        out_specs=pl.BlockSpec((_RB, 16), lambda i: (i, 0)),
        out_shape=jax.ShapeDtypeStruct((N, OUT_CH), jnp.float32),
        name="tc_gat_out",
    )(acc2, b2)


@jax.jit
def kernel(x, edge_index, W1, att_src1, att_dst1, bias1,
           W2, att_src2, att_dst2, bias2):
    loop = jnp.arange(N, dtype=jnp.int32)
    npad = E_PAD - E_TOT
    src = jnp.concatenate([edge_index[0].astype(jnp.int32), loop,
                           jnp.zeros((npad,), jnp.int32)])
    pad_dst = DUMMY + (jnp.arange(npad, dtype=jnp.int32) % (NP - N))
    dst = jnp.concatenate([edge_index[1].astype(jnp.int32), loop, pad_dst])
    src = src.reshape(NW * BLOCKS_PER_W, EB)
    dst = dst.reshape(NW * BLOCKS_PER_W, EB)

    P1, A1 = _tc1(x, W1, att_src1, att_dst1)
    acc1 = _sc_edge_l1(src, dst, P1, A1)
    P2, A2 = _tc2(acc1, bias1, W2, att_src2, att_dst2)
    acc2 = _sc_edge_l2(src, dst, P2, A2)
    return _tc3(acc2, bias2)


# speedup vs baseline: 1.1662x; 1.1662x over previous
import dataclasses
import functools

import jax
import jax.numpy as jnp
from jax import lax
from jax.experimental import pallas as pl
from jax.experimental.pallas import tpu as pltpu
from jax.experimental.pallas import tpu_sc as plsc

N = 10000
E = 320000
IN_CH = 128
HID = 8
HEADS = 8
OUT_CH = 16

NC = 2
NS = 16
LANES = 16

EB = 128
NW = NC * NS
E_TOT = E + N
BLOCKS_PER_W = 82
E_PAD = BLOCKS_PER_W * NW * EB
ROWS_PER_SUB = 632
NP = ROWS_PER_SUB * NS
DUMMY = N

_vmesh = plsc.VectorSubcoreMesh(
    core_axis_name="c", subcore_axis_name="s", num_cores=NC, num_subcores=NS)

_sc_params = pltpu.CompilerParams(use_tc_tiling_on_sc=False)
if "needs_layout_passes" in pltpu.CompilerParams.__dataclass_fields__:
    _sc_params = dataclasses.replace(_sc_params, needs_layout_passes=False)


def _make_sc_edge(wp, wa, npairs, expand, name):
    @functools.partial(
        pl.kernel,
        out_type=jax.ShapeDtypeStruct((NC, NP, wp), jnp.float32),
        mesh=_vmesh,
        scratch_types=[
            pltpu.VMEM((BLOCKS_PER_W, EB), jnp.int32),
            pltpu.VMEM((BLOCKS_PER_W, EB), jnp.int32),
            pltpu.VMEM((EB, wp), jnp.float32),
            pltpu.VMEM((EB, wa), jnp.float32),
            pltpu.VMEM((EB, wp), jnp.float32),
            pltpu.VMEM((EB, wa), jnp.float32),
            pltpu.VMEM((EB, wp), jnp.float32),
            pltpu.VMEM((EB, wp), jnp.float32),
            pltpu.VMEM_SHARED((NP, wp), jnp.float32),
            pltpu.SemaphoreType.DMA,
            pltpu.SemaphoreType.DMA,
        ],
        compiler_params=_sc_params,
        name=name,
    )
    def sc_edge(src_h, dst_h, p_h, a_h, out_h,
                src_i, dst_i, prow0, arow0, prow1, arow1, pay0, pay1,
                accum, sem0, sem1):
        cid = lax.axis_index("c")
        sid = lax.axis_index("s")
        wid = cid * NS + sid

        @pl.loop(0, EB)
        def _(r):
            for k in range(wp // LANES):
                prow0[r, pl.ds(k * LANES, LANES)] = jnp.zeros((LANES,),
                                                              jnp.float32)

        base_row = sid * ROWS_PER_SUB
        for off, nrows in ((0, 128), (128, 128), (256, 128), (384, 128),
                           (512, 120)):
            pltpu.sync_copy(prow0.at[pl.ds(0, nrows)],
                            accum.at[pl.ds(base_row + off, nrows)])
        plsc.subcore_barrier()

        pltpu.sync_copy(src_h.at[pl.ds(wid * BLOCKS_PER_W, BLOCKS_PER_W)],
                        src_i)
        pltpu.sync_copy(dst_h.at[pl.ds(wid * BLOCKS_PER_W, BLOCKS_PER_W)],
                        dst_i)

        score_off = npairs * LANES
        lane = lax.iota(jnp.int32, LANES) // 8
        lanes = [lane + 2 * p for p in range(npairs)]

        def compute(prow, arow, pay):
            @plsc.parallel_loop(0, EB, unroll=4)
            def _(e):
                s = (prow[e, pl.ds(score_off, LANES)]
                     + arow[e, pl.ds(0, LANES)])
                s = jnp.maximum(s, 0.2 * s)
                w = jnp.exp(s)
                pay[e, pl.ds(score_off, LANES)] = w
                if expand:
                    for p in range(npairs):
                        wexp = plsc.load_gather(
                            pay.at[e], [lanes[p] + score_off])
                        pay[e, pl.ds(p * LANES, LANES)] = (
                            prow[e, pl.ds(p * LANES, LANES)] * wexp)
                else:
                    for p in range(npairs):
                        pay[e, pl.ds(p * LANES, LANES)] = (
                            prow[e, pl.ds(p * LANES, LANES)] * w)

        def gather_start(b, prow, arow, sem):
            pltpu.async_copy(p_h.at[src_i.at[b]], prow, sem)
            pltpu.async_copy(a_h.at[dst_i.at[b]], arow, sem)

        def gather_wait(b, prow, arow, sem):
            pltpu.make_async_copy(p_h.at[src_i.at[b]], prow, sem).wait()
            pltpu.make_async_copy(a_h.at[dst_i.at[b]], arow, sem).wait()

        def halfstep(b, prow, arow, pay, sem):
            gather_wait(b, prow, arow, sem)
            compute(prow, arow, pay)
            pltpu.sync_copy(pay, accum.at[dst_i.at[b]], add=True)
            gather_start(jnp.minimum(b + 2, BLOCKS_PER_W - 1), prow, arow,
                         sem)

        gather_start(0, prow0, arow0, sem0)
        gather_start(1, prow1, arow1, sem1)

        @pl.loop(0, BLOCKS_PER_W, step=2)
        def _(b):
            halfstep(b, prow0, arow0, pay0, sem0)
            halfstep(b + 1, prow1, arow1, pay1, sem1)

        gather_wait(0, prow0, arow0, sem0)
        gather_wait(1, prow1, arow1, sem1)

        plsc.subcore_barrier()
        pltpu.sync_copy(accum.at[pl.ds(base_row, ROWS_PER_SUB)],
                        out_h.at[cid, pl.ds(base_row, ROWS_PER_SUB)])

    return sc_edge


_sc_edge_l1 = _make_sc_edge(wp=80, wa=16, npairs=4, expand=True,
                            name="sc_gat_edges_l1")
_sc_edge_l2 = _make_sc_edge(wp=32, wa=16, npairs=1, expand=False,
                            name="sc_gat_edges_l2")


_RB = 1000
_GRID = N // _RB


def _tc1_body(x_ref, w_ref, asr_ref, adr_ref, p_ref, a_ref):
    h = jnp.dot(x_ref[...], w_ref[...], preferred_element_type=jnp.float32)
    hr = h.reshape(_RB, HEADS, HID)
    asrc = (hr * asr_ref[...][None]).sum(-1)
    adst = (hr * adr_ref[...][None]).sum(-1)
    zero8 = jnp.zeros((_RB, 8), jnp.float32)
    p_ref[...] = jnp.concatenate([h, asrc, zero8], axis=1)
    a_ref[...] = jnp.concatenate([adst, zero8], axis=1)


def _tc1(x, W1, as1, ad1):
    return pl.pallas_call(
        _tc1_body,
        grid=(_GRID,),
        in_specs=[
            pl.BlockSpec((_RB, IN_CH), lambda i: (i, 0)),
            pl.BlockSpec((IN_CH, HEADS * HID), lambda i: (0, 0)),
            pl.BlockSpec((HEADS, HID), lambda i: (0, 0)),
            pl.BlockSpec((HEADS, HID), lambda i: (0, 0)),
        ],
        out_specs=[
            pl.BlockSpec((_RB, 80), lambda i: (i, 0)),
            pl.BlockSpec((_RB, 16), lambda i: (i, 0)),
        ],
        out_shape=[
            jax.ShapeDtypeStruct((N, 80), jnp.float32),
            jax.ShapeDtypeStruct((N, 16), jnp.float32),
        ],
        name="tc_gat_dense1",
    )(x, W1, as1, ad1)


def _tc2_body(acc_ref, b1_ref, w2_ref, as2_ref, ad2_ref, p_ref, a_ref):
    acc = acc_ref[0] + acc_ref[1]
    den = acc[:, 64:72]
    msg = acc[:, 0:64].reshape(_RB, HEADS, HID)
    o1 = msg / (den[:, :, None] + 1e-16)
    z = o1.reshape(_RB, HEADS * HID) + b1_ref[...][None]
    act = jnp.where(z > 0, z, jnp.exp(jnp.minimum(z, 0.0)) - 1.0)
    h2 = jnp.dot(act, w2_ref[...], preferred_element_type=jnp.float32)
    asrc2 = (h2 * as2_ref[...]).sum(-1, keepdims=True)
    adst2 = (h2 * ad2_ref[...]).sum(-1, keepdims=True)
    p_ref[...] = jnp.concatenate(
        [h2, jnp.broadcast_to(asrc2, (_RB, 16))], axis=1)
    a_ref[...] = jnp.broadcast_to(adst2, (_RB, 16))


def _tc2(acc1, b1, W2, as2, ad2):
    return pl.pallas_call(
        _tc2_body,
        grid=(_GRID,),
        in_specs=[
            pl.BlockSpec((2, _RB, 80), lambda i: (0, i, 0)),
            pl.BlockSpec((HEADS * HID,), lambda i: (0,)),
            pl.BlockSpec((HEADS * HID, OUT_CH), lambda i: (0, 0)),
            pl.BlockSpec((1, OUT_CH), lambda i: (0, 0)),
            pl.BlockSpec((1, OUT_CH), lambda i: (0, 0)),
        ],
        out_specs=[
            pl.BlockSpec((_RB, 32), lambda i: (i, 0)),
            pl.BlockSpec((_RB, 16), lambda i: (i, 0)),
        ],
        out_shape=[
            jax.ShapeDtypeStruct((N, 32), jnp.float32),
            jax.ShapeDtypeStruct((N, 16), jnp.float32),
        ],
        name="tc_gat_dense2",
    )(acc1, b1, W2, as2, ad2)


def _tc3_body(acc_ref, b2_ref, o_ref):
    acc = acc_ref[0] + acc_ref[1]
    den = acc[:, 16:17]
    o = acc[:, 0:16] / (den + 1e-16) + b2_ref[...][None]
    m = o.max(axis=1, keepdims=True)
    s = o - m
    o_ref[...] = s - jnp.log(jnp.sum(jnp.exp(s), axis=1, keepdims=True))


def _tc3(acc2, b2):
    return pl.pallas_call(
        _tc3_body,
        grid=(_GRID,),
        in_specs=[
            pl.BlockSpec((2, _RB, 32), lambda i: (0, i, 0)),
            pl.BlockSpec((OUT_CH,), lambda i: (0,)),
        ],
        out_specs=pl.BlockSpec((_RB, 16), lambda i: (i, 0)),
        out_shape=jax.ShapeDtypeStruct((N, OUT_CH), jnp.float32),
        name="tc_gat_out",
    )(acc2, b2)


@jax.jit
def kernel(x, edge_index, W1, att_src1, att_dst1, bias1,
           W2, att_src2, att_dst2, bias2):
    loop = jnp.arange(N, dtype=jnp.int32)
    npad = E_PAD - E_TOT
    src = jnp.concatenate([edge_index[0].astype(jnp.int32), loop,
                           jnp.zeros((npad,), jnp.int32)])
    pad_dst = DUMMY + (jnp.arange(npad, dtype=jnp.int32) % (NP - N))
    dst = jnp.concatenate([edge_index[1].astype(jnp.int32), loop, pad_dst])
    src = src.reshape(NW * BLOCKS_PER_W, EB)
    dst = dst.reshape(NW * BLOCKS_PER_W, EB)

    P1, A1 = _tc1(x, W1, att_src1, att_dst1)
    acc1 = _sc_edge_l1(src, dst, P1, A1)
    P2, A2 = _tc2(acc1, bias1, W2, att_src2, att_dst2)
    acc2 = _sc_edge_l2(src, dst, P2, A2)
    return _tc3(acc2, bias2)
